# baseline (speedup 1.0000x reference)
import functools

import jax
import jax.numpy as jnp
from jax import lax
from jax.experimental import pallas as pl
from jax.experimental.pallas import tpu as pltpu
from jax.experimental.pallas import tpu_sc as plsc

N = 100000
D = 128
NW = 32
WPW = 3136
CH = 112
NG = CH // 16

R0, R1, R2, R3 = 119, 5 * 12, 12 * 10, 6 * 6 * 2 * 2


def _build_tables_body(e0, e1, e2, e3, e4, e5, e6, e7, e8, o0, o1, o2, o3):
    o0[...] = e0[...]
    o1[...] = (e1[...][:, None, :] + e2[...][None, :, :]).reshape(R1, D)
    o2[...] = (e3[...][:, None, :] + e4[...][None, :, :]).reshape(R2, D)
    t56 = (e5[...][:, None, :] + e6[...][None, :, :]).reshape(36, D)
    t78 = (e7[...][:, None, :] + e8[...][None, :, :]).reshape(4, D)
    o3[...] = (t56[:, None, :] + t78[None, :, :]).reshape(R3, D)


_build_tables = pl.pallas_call(
    _build_tables_body,
    out_shape=(
        jax.ShapeDtypeStruct((R0, D), jnp.float32),
        jax.ShapeDtypeStruct((R1, D), jnp.float32),
        jax.ShapeDtypeStruct((R2, D), jnp.float32),
        jax.ShapeDtypeStruct((R3, D), jnp.float32),
    ),
)


def _sc_main(xf, t0, t1, t2, t3):
    mesh = plsc.VectorSubcoreMesh(core_axis_name="c", subcore_axis_name="s")

    @functools.partial(
        pl.kernel,
        mesh=mesh,
        out_type=jax.ShapeDtypeStruct((N * D,), jnp.float32),
        compiler_params=pltpu.CompilerParams(needs_layout_passes=False),
        scratch_types=[
            pltpu.VMEM((R0 * D,), jnp.float32),
            pltpu.VMEM((R1 * D,), jnp.float32),
            pltpu.VMEM((R2 * D,), jnp.float32),
            pltpu.VMEM((R3 * D,), jnp.float32),
            [pltpu.VMEM((CH * 9,), jnp.int32) for _ in range(2)],
            [pltpu.VMEM((CH * D,), jnp.float32) for _ in range(2)],
            [pltpu.SemaphoreType.DMA for _ in range(2)],
            [pltpu.SemaphoreType.DMA for _ in range(2)],
        ],
    )
    def k(x_hbm, t0_hbm, t1_hbm, t2_hbm, t3_hbm, out_hbm,
          tb0, tb1, tb2, tb3, xs, os, xsem, osem):
        wid = lax.axis_index("s") * 2 + lax.axis_index("c")
        base_w = wid * WPW
        rows_w = jnp.minimum(N - base_w, WPW)
        nch = (rows_w + (CH - 1)) // CH

        pltpu.sync_copy(t0_hbm, tb0)
        pltpu.sync_copy(t1_hbm, tb1)
        pltpu.sync_copy(t2_hbm, tb2)
        pltpu.sync_copy(t3_hbm, tb3)

        iot = lax.iota(jnp.int32, 16)
        bcast_dn = lax.GatherDimensionNumbers(
            offset_dims=(), collapsed_slice_dims=(0,), start_index_map=(0,))

        def lane_bcast(v, i):
            idx = jnp.full((16, 1), i, jnp.int32)
            return lax.gather(v, idx, bcast_dn, slice_sizes=(1,),
                              mode=lax.GatherScatterMode.PROMISE_IN_BOUNDS)

        def chunk_base(ci):
            return jnp.minimum(base_w + ci * CH, N - CH)

        def fire_x(ci, s):
            pltpu.async_copy(
                x_hbm.at[pl.ds(chunk_base(ci) * 9, CH * 9)], xs[s], xsem[s])

        def wait_x(s):
            pltpu.make_async_copy(
                x_hbm.at[pl.ds(0, CH * 9)], xs[s], xsem[s]).wait()

        def wait_out(s):
            pltpu.make_async_copy(
                os[s], out_hbm.at[pl.ds(0, CH * D)], osem[s]).wait()

        def compute(ci, s):
            base = chunk_base(ci)

            @plsc.parallel_loop(0, NG)
            def group_body(g):
                rl = iot + g * 16
                ax = rl * 9
                xv = [plsc.load_gather(xs[s], [ax + j]) for j in range(9)]
                a0 = xv[0] * D
                a1 = (xv[1] * 12 + xv[2]) * D
                a2 = (xv[3] * 10 + xv[4]) * D
                a3 = (xv[5] * 24 + xv[6] * 4 + xv[7] * 2 + xv[8]) * D
                ro = g * (16 * D)
                for i in range(16):
                    b0 = lane_bcast(a0, i)
                    b1 = lane_bcast(a1, i)
                    b2 = lane_bcast(a2, i)
                    b3 = lane_bcast(a3, i)
                    for c0 in range(0, D, 16):
                        cv = iot + c0
                        v = plsc.load_gather(tb0, [b0 + cv])
                        v = v + plsc.load_gather(tb1, [b1 + cv])
                        v = v + plsc.load_gather(tb2, [b2 + cv])
                        v = v + plsc.load_gather(tb3, [b3 + cv])
                        os[s][pl.ds(ro + i * D + c0, 16)] = v

            pltpu.async_copy(os[s], out_hbm.at[pl.ds(base * D, CH * D)],
                             osem[s])

        fire_x(0, 0)

        def pair_body(p, carry):
            ci = p * 2

            @pl.when(ci + 1 < nch)
            def _():
                fire_x(ci + 1, 1)

            wait_x(0)

            @pl.when(p > 0)
            def _():
                wait_out(0)

            compute(ci, 0)

            @pl.when(ci + 2 < nch)
            def _():
                fire_x(ci + 2, 0)

            @pl.when(ci + 1 < nch)
            def _():
                wait_x(1)

                @pl.when(p > 0)
                def _():
                    wait_out(1)

                compute(ci + 1, 1)

            return carry

        lax.fori_loop(0, (nch + 1) // 2, pair_body, 0)
        wait_out(0)

        @pl.when(nch > 1)
        def _():
            wait_out(1)

    return k(xf, t0, t1, t2, t3)


def kernel(x, emb0, emb1, emb2, emb3, emb4, emb5, emb6, emb7, emb8):
    t0, t1, t2, t3 = _build_tables(emb0, emb1, emb2, emb3, emb4,
                                   emb5, emb6, emb7, emb8)
    out = _sc_main(x.reshape(N * 9), t0.reshape(R0 * D), t1.reshape(R1 * D),
                   t2.reshape(R2 * D), t3.reshape(R3 * D))
    return out.reshape(N, D)


# speedup vs baseline: 1.1444x; 1.1444x over previous
import functools

import jax
import jax.numpy as jnp
from jax import lax
from jax.experimental import pallas as pl
from jax.experimental.pallas import tpu as pltpu
from jax.experimental.pallas import tpu_sc as plsc

N = 100000
D = 128
NW = 32
WPW = 3136
CH = 160
NG = CH // 16

R0, R1, R2, R3 = 119, 5 * 12, 12 * 10, 6 * 6 * 2 * 2


def _build_tables_body(e0, e1, e2, e3, e4, e5, e6, e7, e8, o0, o1, o2, o3):
    o0[...] = e0[...]
    o1[...] = (e1[...][:, None, :] + e2[...][None, :, :]).reshape(R1, D)
    o2[...] = (e3[...][:, None, :] + e4[...][None, :, :]).reshape(R2, D)
    t56 = (e5[...][:, None, :] + e6[...][None, :, :]).reshape(36, D)
    t78 = (e7[...][:, None, :] + e8[...][None, :, :]).reshape(4, D)
    o3[...] = (t56[:, None, :] + t78[None, :, :]).reshape(R3, D)


_build_tables = pl.pallas_call(
    _build_tables_body,
    out_shape=(
        jax.ShapeDtypeStruct((R0, D), jnp.float32),
        jax.ShapeDtypeStruct((R1, D), jnp.float32),
        jax.ShapeDtypeStruct((R2, D), jnp.float32),
        jax.ShapeDtypeStruct((R3, D), jnp.float32),
    ),
)


def _sc_main(xf, t0, t1, t2, t3):
    mesh = plsc.VectorSubcoreMesh(core_axis_name="c", subcore_axis_name="s")

    @functools.partial(
        pl.kernel,
        mesh=mesh,
        out_type=jax.ShapeDtypeStruct((N * D,), jnp.float32),
        compiler_params=pltpu.CompilerParams(needs_layout_passes=False),
        scratch_types=[
            pltpu.VMEM((R0 * D,), jnp.float32),
            pltpu.VMEM((R1 * D,), jnp.float32),
            pltpu.VMEM((R2 * D,), jnp.float32),
            pltpu.VMEM((R3 * D,), jnp.float32),
            [pltpu.VMEM((CH * 9,), jnp.int32) for _ in range(2)],
            [pltpu.VMEM((CH * D,), jnp.float32) for _ in range(2)],
            [pltpu.SemaphoreType.DMA for _ in range(2)],
            [pltpu.SemaphoreType.DMA for _ in range(2)],
        ],
    )
    def k(x_hbm, t0_hbm, t1_hbm, t2_hbm, t3_hbm, out_hbm,
          tb0, tb1, tb2, tb3, xs, os, xsem, osem):
        wid = lax.axis_index("s") * 2 + lax.axis_index("c")
        base_w = wid * WPW
        rows_w = jnp.minimum(N - base_w, WPW)
        nch = (rows_w + (CH - 1)) // CH

        pltpu.sync_copy(t0_hbm, tb0)
        pltpu.sync_copy(t1_hbm, tb1)
        pltpu.sync_copy(t2_hbm, tb2)
        pltpu.sync_copy(t3_hbm, tb3)

        iot = lax.iota(jnp.int32, 16)
        bcast_dn = lax.GatherDimensionNumbers(
            offset_dims=(), collapsed_slice_dims=(0,), start_index_map=(0,))

        def lane_bcast(v, i):
            idx = jnp.full((16, 1), i, jnp.int32)
            return lax.gather(v, idx, bcast_dn, slice_sizes=(1,),
                              mode=lax.GatherScatterMode.PROMISE_IN_BOUNDS)

        def chunk_base(ci):
            return jnp.minimum(base_w + ci * CH, N - CH)

        def fire_x(ci, s):
            pltpu.async_copy(
                x_hbm.at[pl.ds(chunk_base(ci) * 9, CH * 9)], xs[s], xsem[s])

        def wait_x(s):
            pltpu.make_async_copy(
                x_hbm.at[pl.ds(0, CH * 9)], xs[s], xsem[s]).wait()

        def wait_out(s):
            pltpu.make_async_copy(
                os[s], out_hbm.at[pl.ds(0, CH * D)], osem[s]).wait()

        def compute(ci, s):
            base = chunk_base(ci)

            @plsc.parallel_loop(0, NG)
            def group_body(g):
                rl = iot + g * 16
                ax = rl * 9
                xv = [plsc.load_gather(xs[s], [ax + j]) for j in range(9)]
                a0 = xv[0] * D
                a1 = (xv[1] * 12 + xv[2]) * D
                a2 = (xv[3] * 10 + xv[4]) * D
                a3 = (xv[5] * 24 + xv[6] * 4 + xv[7] * 2 + xv[8]) * D
                ro = g * (16 * D)
                for i in range(16):
                    b0 = lane_bcast(a0, i)
                    b1 = lane_bcast(a1, i)
                    b2 = lane_bcast(a2, i)
                    b3 = lane_bcast(a3, i)
                    for c0 in range(0, D, 16):
                        cv = iot + c0
                        v01 = (plsc.load_gather(tb0, [b0 + cv])
                               + plsc.load_gather(tb1, [b1 + cv]))
                        v23 = (plsc.load_gather(tb2, [b2 + cv])
                               + plsc.load_gather(tb3, [b3 + cv]))
                        os[s][pl.ds(ro + i * D + c0, 16)] = v01 + v23

            pltpu.async_copy(os[s], out_hbm.at[pl.ds(base * D, CH * D)],
                             osem[s])

        fire_x(0, 0)

        def pair_body(p, carry):
            ci = p * 2

            @pl.when(ci + 1 < nch)
            def _():
                fire_x(ci + 1, 1)

            wait_x(0)

            @pl.when(p > 0)
            def _():
                wait_out(0)

            compute(ci, 0)

            @pl.when(ci + 2 < nch)
            def _():
                fire_x(ci + 2, 0)

            @pl.when(ci + 1 < nch)
            def _():
                wait_x(1)

                @pl.when(p > 0)
                def _():
                    wait_out(1)

                compute(ci + 1, 1)

            return carry

        lax.fori_loop(0, (nch + 1) // 2, pair_body, 0)
        wait_out(0)

        @pl.when(nch > 1)
        def _():
            wait_out(1)

    return k(xf, t0, t1, t2, t3)


def kernel(x, emb0, emb1, emb2, emb3, emb4, emb5, emb6, emb7, emb8):
    t0, t1, t2, t3 = _build_tables(emb0, emb1, emb2, emb3, emb4,
                                   emb5, emb6, emb7, emb8)
    out = _sc_main(x.reshape(N * 9), t0.reshape(R0 * D), t1.reshape(R1 * D),
                   t2.reshape(R2 * D), t3.reshape(R3 * D))
    return out.reshape(N, D)
